# baseline (speedup 1.0000x reference)
import functools

import jax
import jax.numpy as jnp
from jax import lax
from jax.experimental import pallas as pl
from jax.experimental.pallas import tpu as pltpu
from jax.experimental.pallas import tpu_sc as plsc

NC, NS, L = 2, 16, 16
D = 33
AK0, NA = 14, 17
H, W = 96, 160
RW = 8
NCH = H // RW
NVEC = RW * W // L
WPR = W // L

_WIN = ((-2, 0.3), (-1, 0.7), (0, 1.0), (1, 0.7), (2, 0.3))


def kernel(x, disp):
    b, c, h, w = x.shape
    mesh = plsc.VectorSubcoreMesh(
        core_axis_name="c", subcore_axis_name="s", num_cores=NC, num_subcores=NS
    )

    @functools.partial(
        pl.kernel,
        out_type=jax.ShapeDtypeStruct((b, c, D, h, w), jnp.float32),
        mesh=mesh,
        compiler_params=pltpu.CompilerParams(needs_layout_passes=False),
        scratch_types=[
            pltpu.VMEM((7, RW, W), jnp.float32),
            pltpu.VMEM((NA, RW, W), jnp.float32),
            pltpu.VMEM((NA, RW, W), jnp.float32),
            pltpu.VMEM((RW, W), jnp.float32),
            pltpu.VMEM((RW, W), jnp.float32),
            pltpu.SemaphoreType.DMA,
            pltpu.SemaphoreType.DMA,
            pltpu.SemaphoreType.DMA,
        ],
    )
    def sck(x_hbm, disp_hbm, out_hbm, zbuf, abuf0, abuf1, dbuf, xbuf, asem0, asem1, zsem):
        wid = lax.axis_index("s") * NC + lax.axis_index("c")

        zvec = jnp.zeros((L,), jnp.float32)

        def zero_zbuf(i, carry):
            r = i // WPR
            col = (i % WPR) * L
            zbuf[r // RW, r % RW, pl.ds(col, L)] = zvec
            return carry

        lax.fori_loop(0, 7 * RW * WPR, zero_zbuf, 0)

        abufs = (abuf0, abuf1)
        lane = lax.iota(jnp.int32, L)
        active_handles = [None, None]
        zero_handles = []

        for bi in range(2):
            ci = wid
            for ch in range(NCH):
                u = ch % 2
                ab = abufs[u]
                r0 = ch * RW

                hd = active_handles[u]
                if hd is not None:
                    hd.wait()

                def zero_ab(i, carry, ab=ab):
                    r = i // WPR
                    col = (i % WPR) * L
                    ab[r // RW, r % RW, pl.ds(col, L)] = zvec
                    return carry

                lax.fori_loop(0, NA * RW * WPR, zero_ab, 0)

                pltpu.sync_copy(disp_hbm.at[bi, 0, pl.ds(r0, RW)], dbuf)
                pltpu.sync_copy(x_hbm.at[bi, ci, pl.ds(r0, RW)], xbuf)

                def scatter_vec(i, carry, ab=ab):
                    r = i // WPR
                    col = (i % WPR) * L
                    dv = dbuf[r, pl.ds(col, L)]
                    xv = xbuf[r, pl.ds(col, L)]
                    j = (dv * 13.0).astype(jnp.int32) + 16
                    kk = j - AK0
                    ridx = jnp.full((L,), r, jnp.int32)
                    cidx = col + lane
                    for dt, wt in _WIN:
                        plsc.store_scatter(ab, [kk + dt, ridx, cidx], xv * wt)
                    return carry

                lax.fori_loop(0, NVEC, scatter_vec, 0)

                active_handles[u] = pltpu.async_copy(
                    ab, out_hbm.at[bi, ci, pl.ds(AK0, NA), pl.ds(r0, RW)],
                    (asem0, asem1)[u]
                )
                zero_handles.append(pltpu.async_copy(
                    zbuf, out_hbm.at[bi, ci, pl.ds(0, 7), pl.ds(r0, RW)], zsem
                ))
                zero_handles.append(pltpu.async_copy(
                    zbuf, out_hbm.at[bi, ci, pl.ds(7, 7), pl.ds(r0, RW)], zsem
                ))
                zero_handles.append(pltpu.async_copy(
                    zbuf.at[pl.ds(0, 2)], out_hbm.at[bi, ci, pl.ds(31, 2), pl.ds(r0, RW)], zsem
                ))

        for hd in active_handles:
            if hd is not None:
                hd.wait()
        for hd in zero_handles:
            hd.wait()

    return sck(x, disp)


# speedup vs baseline: 1.6169x; 1.6169x over previous
import functools

import jax
import jax.numpy as jnp
from jax import lax
from jax.experimental import pallas as pl
from jax.experimental.pallas import tpu as pltpu
from jax.experimental.pallas import tpu_sc as plsc

NC, NS, L = 2, 16, 16
D = 33
AK0, NA = 14, 17
H, W = 96, 160
RW = 8
NCH = H // RW
NVEC = RW * W // L
WPR = W // L

_WIN = ((-2, 0.3), (-1, 0.7), (0, 1.0), (1, 0.7), (2, 0.3))


def kernel(x, disp):
    b, c, h, w = x.shape
    mesh = plsc.VectorSubcoreMesh(
        core_axis_name="c", subcore_axis_name="s", num_cores=NC, num_subcores=NS
    )

    @functools.partial(
        pl.kernel,
        out_type=jax.ShapeDtypeStruct((b, c, D, h, w), jnp.float32),
        mesh=mesh,
        compiler_params=pltpu.CompilerParams(needs_layout_passes=False),
        scratch_types=[
            pltpu.VMEM((7, RW, W), jnp.float32),
            pltpu.VMEM((NA, RW, W), jnp.float32),
            pltpu.VMEM((NA, RW, W), jnp.float32),
            pltpu.VMEM((RW, W), jnp.float32),
            pltpu.VMEM((RW, W), jnp.float32),
            pltpu.VMEM((RW, W), jnp.float32),
            pltpu.VMEM((RW, W), jnp.float32),
            pltpu.SemaphoreType.DMA,
            pltpu.SemaphoreType.DMA,
            pltpu.SemaphoreType.DMA,
            pltpu.SemaphoreType.DMA,
            pltpu.SemaphoreType.DMA,
        ],
    )
    def sck(x_hbm, disp_hbm, out_hbm, zbuf, abuf0, abuf1, dbuf0, dbuf1,
            xbuf0, xbuf1, asem0, asem1, isem0, isem1, zsem):
        wid = lax.axis_index("s") * NC + lax.axis_index("c")

        zvec = jnp.zeros((L,), jnp.float32)
        lane = lax.iota(jnp.int32, L)

        def zero_buf(buf, planes):
            def outer(a, carry):
                def inner(r, carry2):
                    for cc in range(WPR):
                        buf[a, r, pl.ds(cc * L, L)] = zvec
                    return carry2
                return lax.fori_loop(0, RW, inner, carry)
            lax.fori_loop(0, planes, outer, 0)

        zero_buf(zbuf, 7)

        abufs = (abuf0, abuf1)
        dbufs = (dbuf0, dbuf1)
        xbufs = (xbuf0, xbuf1)
        asems = (asem0, asem1)
        isems = (isem0, isem1)

        total = 2 * NCH
        chunk_pos = [(g // NCH, (g % NCH) * RW) for g in range(total)]

        def stage(g, u):
            bi, r0 = chunk_pos[g]
            return (
                pltpu.async_copy(disp_hbm.at[bi, 0, pl.ds(r0, RW)], dbufs[u], isems[u]),
                pltpu.async_copy(x_hbm.at[bi, wid, pl.ds(r0, RW)], xbufs[u], isems[u]),
            )

        active_handles = [None, None]
        stage_handles = [None, None]
        zero_handles = []

        stage_handles[0] = stage(0, 0)

        for g in range(total):
            u = g % 2
            bi, r0 = chunk_pos[g]
            ab = abufs[u]

            hd = active_handles[u]
            if hd is not None:
                hd.wait()

            zero_buf(ab, NA)

            for sh in stage_handles[u]:
                sh.wait()
            if g + 1 < total:
                stage_handles[1 - u] = stage(g + 1, 1 - u)

            db, xb = dbufs[u], xbufs[u]

            def scatter_vec(i, carry, ab=ab, db=db, xb=xb):
                r = i // WPR
                col = (i % WPR) * L
                dv = db[r, pl.ds(col, L)]
                xv = xb[r, pl.ds(col, L)]
                j = (dv * 13.0).astype(jnp.int32) + 16
                kk = j - AK0
                ridx = jnp.full((L,), r, jnp.int32)
                cidx = col + lane
                for dt, wt in _WIN:
                    plsc.store_scatter(ab, [kk + dt, ridx, cidx], xv * wt)
                return carry

            lax.fori_loop(0, NVEC, scatter_vec, 0)

            active_handles[u] = pltpu.async_copy(
                ab, out_hbm.at[bi, wid, pl.ds(AK0, NA), pl.ds(r0, RW)], asems[u]
            )
            zero_handles.append(pltpu.async_copy(
                zbuf, out_hbm.at[bi, wid, pl.ds(0, 7), pl.ds(r0, RW)], zsem
            ))
            zero_handles.append(pltpu.async_copy(
                zbuf, out_hbm.at[bi, wid, pl.ds(7, 7), pl.ds(r0, RW)], zsem
            ))
            zero_handles.append(pltpu.async_copy(
                zbuf.at[pl.ds(0, 2)], out_hbm.at[bi, wid, pl.ds(31, 2), pl.ds(r0, RW)], zsem
            ))

        for hd in active_handles:
            if hd is not None:
                hd.wait()
        for hd in zero_handles:
            hd.wait()

    return sck(x, disp)


# speedup vs baseline: 1.6283x; 1.0071x over previous
import functools

import jax
import jax.numpy as jnp
from jax import lax
from jax.experimental import pallas as pl
from jax.experimental.pallas import tpu as pltpu
from jax.experimental.pallas import tpu_sc as plsc

NC, NS, L = 2, 16, 16
D = 33
AK0, NA = 14, 17
H, W = 96, 160
RW = 8
NCH = H // RW
NVEC = RW * W // L
WPR = W // L

_WIN = ((-2, 0.3), (-1, 0.7), (0, 1.0), (1, 0.7), (2, 0.3))


def kernel(x, disp):
    b, c, h, w = x.shape
    mesh = plsc.VectorSubcoreMesh(
        core_axis_name="c", subcore_axis_name="s", num_cores=NC, num_subcores=NS
    )

    @functools.partial(
        pl.kernel,
        out_type=jax.ShapeDtypeStruct((b, c, D, h, w), jnp.float32),
        mesh=mesh,
        compiler_params=pltpu.CompilerParams(needs_layout_passes=False),
        scratch_types=[
            pltpu.VMEM((14, RW, W), jnp.float32),
            pltpu.VMEM((NA, RW, W), jnp.float32),
            pltpu.VMEM((NA, RW, W), jnp.float32),
            pltpu.VMEM((RW, W), jnp.float32),
            pltpu.VMEM((RW, W), jnp.float32),
            pltpu.VMEM((RW, W), jnp.float32),
            pltpu.VMEM((RW, W), jnp.float32),
            pltpu.VMEM((NVEC * L,), jnp.int32),
            pltpu.VMEM((NVEC * L,), jnp.int32),
            pltpu.SemaphoreType.DMA,
            pltpu.SemaphoreType.DMA,
            pltpu.SemaphoreType.DMA,
            pltpu.SemaphoreType.DMA,
            pltpu.SemaphoreType.DMA,
        ],
    )
    def sck(x_hbm, disp_hbm, out_hbm, zbuf, abuf0, abuf1, dbuf0, dbuf1,
            xbuf0, xbuf1, okbuf0, okbuf1, asem0, asem1, isem0, isem1, zsem):
        wid = lax.axis_index("s") * NC + lax.axis_index("c")

        zvec = jnp.zeros((L,), jnp.float32)
        lane = lax.iota(jnp.int32, L)

        def zero_buf(buf, planes):
            def outer(a, carry):
                def inner(r, carry2):
                    for cc in range(WPR):
                        buf[a, r, pl.ds(cc * L, L)] = zvec
                    return carry2
                return lax.fori_loop(0, RW, inner, carry)
            lax.fori_loop(0, planes, outer, 0)

        zero_buf(zbuf, 14)
        zero_buf(abuf0, NA)
        zero_buf(abuf1, NA)

        abufs = (abuf0, abuf1)
        okbufs = (okbuf0, okbuf1)
        dbufs = (dbuf0, dbuf1)
        xbufs = (xbuf0, xbuf1)
        asems = (asem0, asem1)
        isems = (isem0, isem1)

        total = 2 * NCH
        chunk_pos = [(g // NCH, (g % NCH) * RW) for g in range(total)]

        def stage(g, u):
            bi, r0 = chunk_pos[g]
            return (
                pltpu.async_copy(disp_hbm.at[bi, 0, pl.ds(r0, RW)], dbufs[u], isems[u]),
                pltpu.async_copy(x_hbm.at[bi, wid, pl.ds(r0, RW)], xbufs[u], isems[u]),
            )

        active_handles = [None, None]
        stage_handles = [None, None]
        zero_handles = []

        stage_handles[0] = stage(0, 0)

        for g in range(total):
            u = g % 2
            bi, r0 = chunk_pos[g]
            ab = abufs[u]

            hd = active_handles[u]
            ok = okbufs[u]
            if hd is not None:
                hd.wait()

                def rezero_vec(i, carry, ab=ab, ok=ok):
                    r = i // WPR
                    col = (i % WPR) * L
                    kk = ok[pl.ds(i * L, L)]
                    ridx = jnp.full((L,), r, jnp.int32)
                    cidx = col + lane
                    for dt, _ in _WIN:
                        plsc.store_scatter(ab, [kk + dt, ridx, cidx], zvec)
                    return carry

                lax.fori_loop(0, NVEC, rezero_vec, 0)

            for sh in stage_handles[u]:
                sh.wait()
            if g + 1 < total:
                stage_handles[1 - u] = stage(g + 1, 1 - u)

            db, xb = dbufs[u], xbufs[u]

            def scatter_vec(i, carry, ab=ab, db=db, xb=xb, ok=ok):
                r = i // WPR
                col = (i % WPR) * L
                dv = db[r, pl.ds(col, L)]
                xv = xb[r, pl.ds(col, L)]
                j = (dv * 13.0).astype(jnp.int32) + 16
                kk = j - AK0
                ok[pl.ds(i * L, L)] = kk
                ridx = jnp.full((L,), r, jnp.int32)
                cidx = col + lane
                for dt, wt in _WIN:
                    plsc.store_scatter(ab, [kk + dt, ridx, cidx], xv * wt)
                return carry

            lax.fori_loop(0, NVEC, scatter_vec, 0)

            active_handles[u] = pltpu.async_copy(
                ab, out_hbm.at[bi, wid, pl.ds(AK0, NA), pl.ds(r0, RW)], asems[u]
            )
            zero_handles.append(pltpu.async_copy(
                zbuf, out_hbm.at[bi, wid, pl.ds(0, 14), pl.ds(r0, RW)], zsem
            ))
            zero_handles.append(pltpu.async_copy(
                zbuf.at[pl.ds(0, 2)], out_hbm.at[bi, wid, pl.ds(31, 2), pl.ds(r0, RW)], zsem
            ))

        for hd in active_handles:
            if hd is not None:
                hd.wait()
        for hd in zero_handles:
            hd.wait()

    return sck(x, disp)
